# baseline (speedup 1.0000x reference)
import functools

import jax
import jax.numpy as jnp
from jax import lax
from jax.experimental import pallas as pl
from jax.experimental.pallas import tpu as pltpu
from jax.experimental.pallas import tpu_sc as plsc

N = 10000
E = 320000
D = 128
DH = D // 2

NC = 2
NS = 16
NW = NC * NS
CHUNK = 80
CPW = 125
E_PAD = NW * CPW * CHUNK
N_PAD = 10112
RPT = N_PAD // NS
GRP = 4
CPT = E_PAD // (NS * CHUNK)
QCPT = 50

_mesh = plsc.VectorSubcoreMesh(core_axis_name="c", subcore_axis_name="s")
_SC_PARAMS = pltpu.CompilerParams(use_tc_tiling_on_sc=False,
                                  needs_layout_passes=False)


@functools.partial(
    pl.kernel,
    out_type=jax.ShapeDtypeStruct((NC, N_PAD, 16), jnp.float32),
    mesh=_mesh,
    compiler_params=_SC_PARAMS,
    scratch_types=[
        pltpu.VMEM((CPW, CHUNK), jnp.int32),
        pltpu.VMEM((CHUNK, 16), jnp.float32),
        pltpu.SemaphoreType.DMA,
        pltpu.VMEM_SHARED((N_PAD, 16), jnp.float32),
    ],
)
def _sc_degree(dst2d, ones_hbm, zeros_hbm, deg_out, dst_idx, ones_v, sem, deg_sh):
    cid = lax.axis_index("c")
    sid = lax.axis_index("s")
    w = cid * NS + sid
    for k in range(7):
        pltpu.sync_copy(zeros_hbm.at[:, pl.ds(0, 16)],
                        deg_sh.at[pl.ds(sid * RPT + k * CHUNK, CHUNK)])
    pltpu.sync_copy(zeros_hbm.at[pl.ds(0, 72), pl.ds(0, 16)],
                    deg_sh.at[pl.ds(sid * RPT + 7 * CHUNK, 72)])
    pltpu.sync_copy(ones_hbm, ones_v)
    pltpu.sync_copy(dst2d.at[pl.ds(w * CPW, CPW)], dst_idx)
    plsc.subcore_barrier()

    def fire(j, carry):
        pltpu.async_copy(ones_v, deg_sh.at[dst_idx.at[j]], sem, add=True)
        return carry

    lax.fori_loop(0, CPW, fire, 0)

    def drain(j, carry):
        pltpu.make_async_copy(ones_hbm, ones_v, sem).wait()
        return carry

    lax.fori_loop(0, CPW, drain, 0)
    plsc.subcore_barrier()
    pltpu.sync_copy(deg_sh.at[pl.ds(sid * RPT, RPT)],
                    deg_out.at[cid, pl.ds(sid * RPT, RPT)])


@functools.partial(
    pl.kernel,
    out_type=jax.ShapeDtypeStruct((N_PAD, D), jnp.float32),
    mesh=_mesh,
    compiler_params=_SC_PARAMS,
    scratch_types=(
        [pltpu.VMEM((QCPT, CHUNK), jnp.int32)] * 2
        + [pltpu.VMEM((CHUNK, DH), jnp.float32)] * (2 * GRP)
        + [pltpu.SemaphoreType.DMA] * (4 * GRP)
        + [pltpu.VMEM_SHARED((N_PAD, DH), jnp.float32)] * 2
    ),
)
def _sc_aggregate(src2d, dst2d, h_hbm, zeros_hbm, agg_out, *refs):
    src_idx, dst_idx = refs[0], refs[1]
    rows = refs[2:2 + 2 * GRP]
    gsem = refs[2 + 2 * GRP:2 + 4 * GRP]
    ssem = refs[2 + 4 * GRP:2 + 6 * GRP]
    h_sh, agg_sh = refs[2 + 6 * GRP], refs[3 + 6 * GRP]
    cid = lax.axis_index("c")
    sid = lax.axis_index("s")
    pltpu.sync_copy(h_hbm.at[pl.ds(sid * RPT, RPT), pl.ds(cid * DH, DH)],
                    h_sh.at[pl.ds(sid * RPT, RPT)])
    for k in range(7):
        pltpu.sync_copy(zeros_hbm,
                        agg_sh.at[pl.ds(sid * RPT + k * CHUNK, CHUNK)])
    pltpu.sync_copy(zeros_hbm.at[pl.ds(0, 72)],
                    agg_sh.at[pl.ds(sid * RPT + 7 * CHUNK, 72)])
    plsc.subcore_barrier()

    NGRP = QCPT // GRP + 1
    _gcnt = lambda g: GRP if g < QCPT // GRP else QCPT - GRP * (QCPT // GRP)

    def fire_gathers(g):
        h0 = (g % 2) * GRP
        for b in range(_gcnt(g)):
            pltpu.async_copy(h_sh.at[src_idx.at[GRP * g + b]],
                             rows[h0 + b], gsem[h0 + b])

    def quarter(qq, carry):
        base = sid * CPT + qq * QCPT
        pltpu.sync_copy(src2d.at[pl.ds(base, QCPT)], src_idx)
        pltpu.sync_copy(dst2d.at[pl.ds(base, QCPT)], dst_idx)
        fire_gathers(0)
        for g in range(NGRP):
            h0 = (g % 2) * GRP
            if g + 1 < NGRP:
                fire_gathers(g + 1)
            for b in range(_gcnt(g)):
                pltpu.make_async_copy(h_sh.at[src_idx.at[GRP * g + b]],
                                      rows[h0 + b], gsem[h0 + b]).wait()
            for b in range(_gcnt(g)):
                pltpu.async_copy(rows[h0 + b],
                                 agg_sh.at[dst_idx.at[GRP * g + b]],
                                 ssem[h0 + b], add=True)
            for b in range(_gcnt(g)):
                pltpu.make_async_copy(zeros_hbm.at[pl.ds(0, CHUNK)],
                                      rows[h0 + b], ssem[h0 + b]).wait()
        return carry

    lax.fori_loop(0, CPT // QCPT, quarter, 0)
    plsc.subcore_barrier()
    pltpu.sync_copy(agg_sh.at[pl.ds(sid * RPT, RPT)],
                    agg_out.at[pl.ds(sid * RPT, RPT), pl.ds(cid * DH, DH)])


def _norm_from_deg(dega, degb):
    deg = dega[:, 0:1] + degb[:, 0:1]
    return jnp.where(deg > 0.0, lax.rsqrt(jnp.maximum(deg, 1.0)), 0.0)


def _tc_scale_body(deg_ref, feats_ref, h_ref):
    norm = _norm_from_deg(deg_ref[0], deg_ref[1])
    h_ref[...] = feats_ref[...] * norm


def _tc_out_body(agg_ref, deg_ref, w_ref, b_ref, out_ref):
    norm = _norm_from_deg(deg_ref[0], deg_ref[1])
    agg = agg_ref[...] * norm
    out_ref[...] = lax.dot_general(
        agg, w_ref[...], (((1,), (1,)), ((), ())),
        preferred_element_type=jnp.float32) + b_ref[...]


_BLK = 2000


def _tc_scale(deg2, feats):
    return pl.pallas_call(
        _tc_scale_body,
        grid=(N // _BLK,),
        in_specs=[
            pl.BlockSpec((NC, _BLK, 16), lambda i: (0, i, 0)),
            pl.BlockSpec((_BLK, D), lambda i: (i, 0)),
        ],
        out_specs=pl.BlockSpec((_BLK, D), lambda i: (i, 0)),
        out_shape=jax.ShapeDtypeStruct((N_PAD, D), jnp.float32),
    )(deg2, feats)


def _tc_linear(agg, deg2, W, b2d):
    return pl.pallas_call(
        _tc_out_body,
        grid=(N // _BLK,),
        in_specs=[
            pl.BlockSpec((_BLK, D), lambda i: (i, 0)),
            pl.BlockSpec((NC, _BLK, 16), lambda i: (0, i, 0)),
            pl.BlockSpec((D, D), lambda i: (0, 0)),
            pl.BlockSpec((1, D), lambda i: (0, 0)),
        ],
        out_specs=pl.BlockSpec((_BLK, D), lambda i: (i, 0)),
        out_shape=jax.ShapeDtypeStruct((N, D), jnp.float32),
    )(agg, deg2, W, b2d)


def kernel(feats, edge_index, W, b):
    src2d = edge_index[0].reshape(E // CHUNK, CHUNK)
    dst2d = edge_index[1].reshape(E // CHUNK, CHUNK)
    ones16 = jnp.ones((CHUNK, 16), jnp.float32)
    zeros64 = jnp.zeros((CHUNK, DH), jnp.float32)

    deg2 = _sc_degree(dst2d, ones16, zeros64)

    h = _tc_scale(deg2, feats)

    agg = _sc_aggregate(src2d, dst2d, h, zeros64)

    out = _tc_linear(agg, deg2, W, b.reshape(1, D))
    return ([out], out)


# speedup vs baseline: 1.0605x; 1.0605x over previous
import functools

import jax
import jax.numpy as jnp
from jax import lax
from jax.experimental import pallas as pl
from jax.experimental.pallas import tpu as pltpu
from jax.experimental.pallas import tpu_sc as plsc

N = 10000
E = 320000
D = 128
DH = D // 2

NC = 2
NS = 16
NW = NC * NS
CHUNK = 80
CPW = 125
E_PAD = NW * CPW * CHUNK
N_PAD = 10112
RPT = N_PAD // NS
GRP = 4
CPT = E_PAD // (NS * CHUNK)
QCPT = 50

_mesh = plsc.VectorSubcoreMesh(core_axis_name="c", subcore_axis_name="s")
_SC_PARAMS = pltpu.CompilerParams(use_tc_tiling_on_sc=False,
                                  needs_layout_passes=False)


@functools.partial(
    pl.kernel,
    out_type=jax.ShapeDtypeStruct((NC * N_PAD, 16), jnp.float32),
    mesh=_mesh,
    compiler_params=_SC_PARAMS,
    scratch_types=[
        pltpu.VMEM((CPW, CHUNK), jnp.int32),
        pltpu.VMEM((CHUNK, 16), jnp.float32),
        pltpu.SemaphoreType.DMA,
        pltpu.VMEM_SHARED((N_PAD, 16), jnp.float32),
    ],
)
def _sc_degree(dst2d, ones_hbm, zeros_hbm, deg_out, dst_idx, ones_v, sem, deg_sh):
    cid = lax.axis_index("c")
    sid = lax.axis_index("s")
    w = cid * NS + sid
    pltpu.sync_copy(zeros_hbm.at[pl.ds(sid * RPT, RPT)],
                    deg_sh.at[pl.ds(sid * RPT, RPT)])
    pltpu.sync_copy(ones_hbm, ones_v)
    pltpu.sync_copy(dst2d.at[pl.ds(w * CPW, CPW)], dst_idx)
    plsc.subcore_barrier()

    def fire(j, carry):
        pltpu.async_copy(ones_v, deg_sh.at[dst_idx.at[j]], sem, add=True)
        return carry

    lax.fori_loop(0, CPW, fire, 0)

    def drain(j, carry):
        pltpu.make_async_copy(ones_hbm, ones_v, sem).wait()
        return carry

    lax.fori_loop(0, CPW, drain, 0)
    plsc.subcore_barrier()
    pltpu.sync_copy(deg_sh.at[pl.ds(sid * RPT, RPT)],
                    deg_out.at[pl.ds(cid * N_PAD + sid * RPT, RPT)])


@functools.partial(
    pl.kernel,
    out_type=jax.ShapeDtypeStruct((N_PAD, D), jnp.float32),
    mesh=_mesh,
    compiler_params=_SC_PARAMS,
    scratch_types=(
        [pltpu.VMEM((QCPT, CHUNK), jnp.int32)] * 2
        + [pltpu.VMEM((CHUNK, DH), jnp.float32)] * (2 * GRP)
        + [pltpu.SemaphoreType.DMA] * (4 * GRP)
        + [pltpu.VMEM_SHARED((N_PAD, DH), jnp.float32)] * 2
    ),
)
def _sc_aggregate(src2d, dst2d, h_hbm, zeros_hbm, agg_out, *refs):
    src_idx, dst_idx = refs[0], refs[1]
    rows = refs[2:2 + 2 * GRP]
    gsem = refs[2 + 2 * GRP:2 + 4 * GRP]
    ssem = refs[2 + 4 * GRP:2 + 6 * GRP]
    h_sh, agg_sh = refs[2 + 6 * GRP], refs[3 + 6 * GRP]
    cid = lax.axis_index("c")
    sid = lax.axis_index("s")
    pltpu.sync_copy(h_hbm.at[pl.ds(sid * RPT, RPT), pl.ds(cid * DH, DH)],
                    h_sh.at[pl.ds(sid * RPT, RPT)])
    pltpu.sync_copy(zeros_hbm.at[pl.ds(sid * RPT, RPT)],
                    agg_sh.at[pl.ds(sid * RPT, RPT)])
    plsc.subcore_barrier()

    NGRP = QCPT // GRP + 1
    _gcnt = lambda g: GRP if g < QCPT // GRP else QCPT - GRP * (QCPT // GRP)

    def fire_gathers(g):
        h0 = (g % 2) * GRP
        for b in range(_gcnt(g)):
            pltpu.async_copy(h_sh.at[src_idx.at[GRP * g + b]],
                             rows[h0 + b], gsem[h0 + b])

    def quarter(qq, carry):
        base = sid * CPT + qq * QCPT
        pltpu.sync_copy(src2d.at[pl.ds(base, QCPT)], src_idx)
        pltpu.sync_copy(dst2d.at[pl.ds(base, QCPT)], dst_idx)
        fire_gathers(0)
        for g in range(NGRP):
            h0 = (g % 2) * GRP
            if g + 1 < NGRP:
                fire_gathers(g + 1)
            for b in range(_gcnt(g)):
                pltpu.make_async_copy(h_sh.at[src_idx.at[GRP * g + b]],
                                      rows[h0 + b], gsem[h0 + b]).wait()
            for b in range(_gcnt(g)):
                pltpu.async_copy(rows[h0 + b],
                                 agg_sh.at[dst_idx.at[GRP * g + b]],
                                 ssem[h0 + b], add=True)
            for b in range(_gcnt(g)):
                pltpu.make_async_copy(zeros_hbm.at[pl.ds(0, CHUNK)],
                                      rows[h0 + b], ssem[h0 + b]).wait()
        return carry

    lax.fori_loop(0, CPT // QCPT, quarter, 0)
    plsc.subcore_barrier()
    pltpu.sync_copy(agg_sh.at[pl.ds(sid * RPT, RPT)],
                    agg_out.at[pl.ds(sid * RPT, RPT), pl.ds(cid * DH, DH)])


def _norm_from_deg(dega, degb):
    deg = dega[:, 0:1] + degb[:, 0:1]
    return jnp.where(deg > 0.0, lax.rsqrt(jnp.maximum(deg, 1.0)), 0.0)


def _tc_scale_body(dega_ref, degb_ref, feats_ref, h_ref):
    norm = _norm_from_deg(dega_ref[...], degb_ref[...])
    h_ref[...] = feats_ref[...] * norm


def _tc_out_body(agg_ref, dega_ref, degb_ref, w_ref, b_ref, out_ref):
    norm = _norm_from_deg(dega_ref[...], degb_ref[...])
    agg = agg_ref[...] * norm
    out_ref[...] = lax.dot_general(
        agg, w_ref[...], (((1,), (1,)), ((), ())),
        preferred_element_type=jnp.float32) + b_ref[...]


_BLK = 2000


def _tc_scale(dega, degb, feats):
    return pl.pallas_call(
        _tc_scale_body,
        grid=(N // _BLK,),
        in_specs=[
            pl.BlockSpec((_BLK, 16), lambda i: (i, 0)),
            pl.BlockSpec((_BLK, 16), lambda i: (i, 0)),
            pl.BlockSpec((_BLK, D), lambda i: (i, 0)),
        ],
        out_specs=pl.BlockSpec((_BLK, D), lambda i: (i, 0)),
        out_shape=jax.ShapeDtypeStruct((N_PAD, D), jnp.float32),
    )(dega, degb, feats)


def _tc_linear(agg, dega, degb, W, b2d):
    return pl.pallas_call(
        _tc_out_body,
        grid=(N // _BLK,),
        in_specs=[
            pl.BlockSpec((_BLK, D), lambda i: (i, 0)),
            pl.BlockSpec((_BLK, 16), lambda i: (i, 0)),
            pl.BlockSpec((_BLK, 16), lambda i: (i, 0)),
            pl.BlockSpec((D, D), lambda i: (0, 0)),
            pl.BlockSpec((1, D), lambda i: (0, 0)),
        ],
        out_specs=pl.BlockSpec((_BLK, D), lambda i: (i, 0)),
        out_shape=jax.ShapeDtypeStruct((N, D), jnp.float32),
    )(agg, dega, degb, W, b2d)


def kernel(feats, edge_index, W, b):
    src2d = edge_index[0].reshape(E // CHUNK, CHUNK)
    dst2d = edge_index[1].reshape(E // CHUNK, CHUNK)
    ones16 = jnp.ones((CHUNK, 16), jnp.float32)
    zeros16 = jnp.zeros((N_PAD, 16), jnp.float32)
    zeros64 = jnp.zeros((N_PAD, DH), jnp.float32)

    deg2 = _sc_degree(dst2d, ones16, zeros16)
    dega, degb = deg2[:N_PAD], deg2[N_PAD:]

    h = _tc_scale(dega, degb, feats)

    agg = _sc_aggregate(src2d, dst2d, h, zeros64)

    out = _tc_linear(agg, dega, degb, W, b.reshape(1, D))
    return ([out], out)
